# Initial kernel scaffold; baseline (speedup 1.0000x reference)
import jax
import jax.numpy as jnp
from jax.experimental import pallas as pl


def kernel(x, edge_index, edge_attr, params):
    raise NotImplementedError("write your pallas kernel here")


# speedup vs baseline: 1.1773x; 1.1773x over previous
import jax
import jax.numpy as jnp
from jax.experimental import pallas as pl


def _lin(x, w, b):
    return x @ w.T + b


def _gcn(x, ei, ew, p):
    n = x.shape[0]
    xw = _lin(x, p['w'], p['b'])
    msg = ew[:, None] * xw[ei[0]]
    aggr = jnp.zeros((n, xw.shape[1]), xw.dtype).at[ei[1]].add(msg) + xw
    out = jax.nn.relu(aggr)
    nrm = jnp.maximum(jnp.linalg.norm(out, axis=-1, keepdims=True), 1e-12)
    return out / nrm


def _gated_restructured(h, row, col, e_nodes, S_prev, c_prev, layer, p,
                        deg_row, deg_col, eps=1e-05):
    E = row.shape[0]
    N = h.shape[0]
    hA = _lin(h, p['A_w'], p['A_b'])
    hB = _lin(h, p['B_w'], p['B_b'])
    eC = _lin(e_nodes, p['C_w'], p['C_b'])
    eD = _lin(e_nodes, p['D_w'], p['D_b'])
    P = hA + eD
    Q = hB + eC
    sum_e = deg_row @ P + deg_col @ Q
    mu = sum_e / E
    cross = jnp.sum(P[row] * Q[col], axis=0)
    sumsq = deg_row @ (P * P) + deg_col @ (Q * Q) + 2.0 * cross
    var = sumsq / E - mu * mu
    s = p['bn_g'] / jnp.sqrt(var + 1e-05)
    Pr = P * s
    Qr = Q * s + (p['bn_b'] - mu * s)
    r = jax.nn.relu(Pr[row] + Qr[col])
    S = jnp.zeros((N, h.shape[1]), h.dtype).at[row].add(r)
    Vh = _lin(h, p['V_w'], p['V_b'])

    def e_pre(k_lo, k_hi):
        e0 = h[row[k_lo:k_hi]] + h[col[k_lo:k_hi]]
        if layer == 2:
            e_prev = jax.nn.sigmoid(e0 + (S_prev if k_lo == 0 else 0.0)) / (c_prev + eps)
        else:
            e_prev = e0
        add = S[k_lo:k_hi] if k_lo == 0 else 0.0
        return e_prev + add

    sig_head = jax.nn.sigmoid(e_pre(0, N))
    sig_tail = jax.nn.sigmoid(e_pre(N, E))
    c = jnp.sum(sig_head, axis=0) + jnp.sum(sig_tail, axis=0)
    t = jnp.sum(sig_head * Vh[col[:N]], axis=0) + jnp.sum(sig_tail * Vh[col[N:]], axis=0)
    h_out = jax.nn.relu(_lin(h, p['U_w'], p['U_b']) + t / (c + eps))
    e_nodes_out = sig_head / (c + eps)
    return h_out, e_nodes_out, S, c


def _cls_kernel(g_ref, w_ref, b_ref, o_ref):
    logits = jnp.sum(g_ref[:] * w_ref[:], axis=1) + b_ref[:]
    m = jnp.max(logits)
    lse = jnp.log(jnp.sum(jnp.exp(logits - m))) + m
    o_ref[:] = logits - lse


def kernel(x, edge_index, edge_attr, params):
    row, col = edge_index[0], edge_index[1]
    N = x.shape[0]
    E = row.shape[0]
    deg_row = jnp.zeros((N,), jnp.float32).at[row].add(1.0)
    deg_col = jnp.zeros((N,), jnp.float32).at[col].add(1.0)
    h = _gcn(x, edge_index, edge_attr, params['gcn1'])
    h = _gcn(h, edge_index, edge_attr, params['gcn2'])
    e_nodes = h[row[:N]] + h[col[:N]]
    h, e_nodes, S1, c1 = _gated_restructured(
        h, row, col, e_nodes, None, None, 1, params['g1'], deg_row, deg_col)
    h, _, _, _ = _gated_restructured(
        h, row, col, e_nodes, S1, c1, 2, params['g2'], deg_row, deg_col)
    g = jnp.mean(h, axis=0)
    w = params['cls']['w']
    b = params['cls']['b']
    out = pl.pallas_call(
        _cls_kernel,
        out_shape=jax.ShapeDtypeStruct((w.shape[0],), jnp.float32),
    )(g[None, :] * jnp.ones((w.shape[0], 1), jnp.float32), w, b)
    return out


# speedup vs baseline: 2.6557x; 2.2557x over previous
import functools

import jax
import jax.numpy as jnp
from jax import lax
from jax.experimental import pallas as pl
from jax.experimental.pallas import tpu as pltpu
from jax.experimental.pallas import tpu_sc as plsc

NCORE = 2
NSUB = 16
NW = NCORE * NSUB
C = 128
D = 128
NSL = D // 16
EPS = 1e-05

_MESH = functools.partial(
    plsc.VectorSubcoreMesh, core_axis_name="c", subcore_axis_name="s",
    num_cores=NCORE, num_subcores=NSUB)


def _wid():
    return lax.axis_index("s") * NCORE + lax.axis_index("c")


def _sl(i):
    return pl.ds(i * 16, 16)


def _sc_gcn(row_p, col_p, ew_p, xw_ext, zeros_big, n, t_chunks, acc_rows):
    def body(row_hbm, col_hbm, ew_hbm, xw_hbm, z_hbm, out_hbm,
             sidx, didx, ewv, rows, acc_sh, sem):
        cid = lax.axis_index("c")
        sid = lax.axis_index("s")
        w = _wid()

        @pl.when(sid == 0)
        def _zero():
            pltpu.sync_copy(z_hbm, acc_sh)

        plsc.subcore_barrier()

        def chunk(i, _):
            base = (w + NW * i) * C
            pltpu.sync_copy(row_hbm.at[pl.ds(base, C)], sidx)
            pltpu.sync_copy(col_hbm.at[pl.ds(base, C)], didx)
            pltpu.sync_copy(ew_hbm.at[pl.ds(base, C)], ewv)
            pltpu.async_copy(xw_hbm.at[sidx], rows, sem).wait()

            def grp(g, _g):
                wvec = ewv[pl.ds(g * 16, 16)]
                for l in range(16):
                    wv = jnp.full((16,), wvec[l], jnp.float32)
                    c = g * 16 + l
                    for s in range(NSL):
                        rows[c, _sl(s)] = rows[c, _sl(s)] * wv
                return 0

            lax.fori_loop(0, C // 16, grp, 0)
            pltpu.sync_copy(rows, acc_sh.at[didx], add=True)
            return 0

        lax.fori_loop(0, t_chunks, chunk, 0)
        plsc.subcore_barrier()

        @pl.when(sid == 0)
        def _out():
            pltpu.sync_copy(acc_sh.at[pl.ds(0, n)], out_hbm.at[cid])

    return pl.kernel(
        body,
        out_type=jax.ShapeDtypeStruct((NCORE, n, D), jnp.float32),
        mesh=_MESH(),
        scratch_types=[
            pltpu.VMEM((C,), jnp.int32),
            pltpu.VMEM((C,), jnp.int32),
            pltpu.VMEM((C,), jnp.float32),
            pltpu.VMEM((C, D), jnp.float32),
            pltpu.VMEM_SHARED((acc_rows, D), jnp.float32),
            pltpu.SemaphoreType.DMA,
        ],
    )(row_p, col_p, ew_p, xw_ext, zeros_big)


def _sc_enodes(row_p, col_p, h_ext, ns_ext):
    n_chunks = ns_ext // C

    def body(row_hbm, col_hbm, h_hbm, out_hbm, ridx, cidx, ra, rb, sem):
        w = _wid()

        def chunk(i, _):
            j = w + NW * i

            @pl.when(j < n_chunks)
            def _do():
                base = j * C
                pltpu.sync_copy(row_hbm.at[pl.ds(base, C)], ridx)
                pltpu.sync_copy(col_hbm.at[pl.ds(base, C)], cidx)
                pltpu.async_copy(h_hbm.at[ridx], ra, sem).wait()
                pltpu.async_copy(h_hbm.at[cidx], rb, sem).wait()

                def edge(c, _c):
                    for s in range(NSL):
                        ra[c, _sl(s)] = ra[c, _sl(s)] + rb[c, _sl(s)]
                    return 0

                lax.fori_loop(0, C, edge, 0)
                pltpu.sync_copy(ra, out_hbm.at[pl.ds(base, C)])

            return 0

        lax.fori_loop(0, (n_chunks + NW - 1) // NW, chunk, 0)

    return pl.kernel(
        body,
        out_type=jax.ShapeDtypeStruct((ns_ext, D), jnp.float32),
        mesh=_MESH(),
        scratch_types=[
            pltpu.VMEM((C,), jnp.int32),
            pltpu.VMEM((C,), jnp.int32),
            pltpu.VMEM((C, D), jnp.float32),
            pltpu.VMEM((C, D), jnp.float32),
            pltpu.SemaphoreType.DMA,
        ],
    )(row_p, col_p, h_ext)


def _sc_stats(row_p, col_p, p_ext, q_ext, t_chunks):
    def body(row_hbm, col_hbm, p_hbm, q_hbm, sum_hbm, sq_hbm,
             ridx, cidx, ra, rb, stage, sem):
        w = _wid()
        zero = jnp.zeros((16,), jnp.float32)
        acc0 = (zero,) * NSL
        acc1 = (zero,) * NSL

        def chunk(i, carry):
            a0, a1 = carry
            base = (w + NW * i) * C
            pltpu.sync_copy(row_hbm.at[pl.ds(base, C)], ridx)
            pltpu.sync_copy(col_hbm.at[pl.ds(base, C)], cidx)
            pltpu.async_copy(p_hbm.at[ridx], ra, sem).wait()
            pltpu.async_copy(q_hbm.at[cidx], rb, sem).wait()

            def edge(c, ec):
                e0, e1 = ec
                n0 = []
                n1 = []
                for s in range(NSL):
                    v = ra[c, _sl(s)] + rb[c, _sl(s)]
                    n0.append(e0[s] + v)
                    n1.append(e1[s] + v * v)
                return (tuple(n0), tuple(n1))

            return lax.fori_loop(0, C, edge, (a0, a1))

        a0, a1 = lax.fori_loop(0, t_chunks, chunk, (acc0, acc1))
        for s in range(NSL):
            stage[_sl(s)] = a0[s]
        pltpu.sync_copy(stage, sum_hbm.at[w])
        for s in range(NSL):
            stage[_sl(s)] = a1[s]
        pltpu.sync_copy(stage, sq_hbm.at[w])

    return pl.kernel(
        body,
        out_type=[jax.ShapeDtypeStruct((NW, D), jnp.float32),
                  jax.ShapeDtypeStruct((NW, D), jnp.float32)],
        mesh=_MESH(),
        scratch_types=[
            pltpu.VMEM((C,), jnp.int32),
            pltpu.VMEM((C,), jnp.int32),
            pltpu.VMEM((C, D), jnp.float32),
            pltpu.VMEM((C, D), jnp.float32),
            pltpu.VMEM((D,), jnp.float32),
            pltpu.SemaphoreType.DMA,
        ],
    )(row_p, col_p, p_ext, q_ext)


def _sc_scatter(row_p, col_p, row_scat, p_ext, q_ext, sqb, zeros_big,
                t_chunks, ns_ext, acc_rows):
    def real_body(row_hbm, col_hbm, rs_hbm, p_hbm, q_hbm, sqb_hbm, z_hbm,
                  out_hbm, ridx, cidx, sidx, ra, rb, coef, acc, sem):
        cid = lax.axis_index("c")
        sid = lax.axis_index("s")
        w = _wid()
        pltpu.sync_copy(sqb_hbm, coef)

        @pl.when(sid == 0)
        def _zero():
            pltpu.sync_copy(z_hbm, acc)

        plsc.subcore_barrier()

        def chunk(i, _):
            base = (w + NW * i) * C
            pltpu.sync_copy(row_hbm.at[pl.ds(base, C)], ridx)
            pltpu.sync_copy(col_hbm.at[pl.ds(base, C)], cidx)
            pltpu.sync_copy(rs_hbm.at[pl.ds(base, C)], sidx)
            pltpu.async_copy(p_hbm.at[ridx], ra, sem).wait()
            pltpu.async_copy(q_hbm.at[cidx], rb, sem).wait()

            def edge(c, _c):
                for s in range(NSL):
                    v = (ra[c, _sl(s)] + rb[c, _sl(s)]) * coef[0, _sl(s)] \
                        + coef[1, _sl(s)]
                    ra[c, _sl(s)] = jnp.maximum(v, 0.0)
                return 0

            lax.fori_loop(0, C, edge, 0)
            pltpu.sync_copy(ra, acc.at[sidx], add=True)
            return 0

        lax.fori_loop(0, t_chunks, chunk, 0)
        plsc.subcore_barrier()

        @pl.when(sid == 0)
        def _out():
            pltpu.sync_copy(acc.at[pl.ds(0, ns_ext)], out_hbm.at[cid])

    return pl.kernel(
        real_body,
        out_type=jax.ShapeDtypeStruct((NCORE, ns_ext, D), jnp.float32),
        mesh=_MESH(),
        scratch_types=[
            pltpu.VMEM((C,), jnp.int32),
            pltpu.VMEM((C,), jnp.int32),
            pltpu.VMEM((C,), jnp.int32),
            pltpu.VMEM((C, D), jnp.float32),
            pltpu.VMEM((C, D), jnp.float32),
            pltpu.VMEM((2, D), jnp.float32),
            pltpu.VMEM_SHARED((acc_rows, D), jnp.float32),
            pltpu.SemaphoreType.DMA,
        ],
    )(row_p, col_p, row_scat, p_ext, q_ext, sqb, zeros_big)


def _sc_ct1(row_p, col_p, h_ext, vh_ext, s2core, t_chunks, ns_ext, e_pad):
    ns_chunks = ns_ext // C

    def body(row_hbm, col_hbm, h_hbm, vh_hbm, s_hbm, cp_hbm, tp_hbm, sig_hbm,
             ridx, cidx, ra, rb, rc, sv0, sv1, stage, sem):
        w = _wid()
        zero = jnp.zeros((16,), jnp.float32)
        one = jnp.full((16,), 1.0, jnp.float32)

        def chunk(i, carry):
            ac, at = carry
            j = w + NW * i
            base = j * C
            pltpu.sync_copy(row_hbm.at[pl.ds(base, C)], ridx)
            pltpu.sync_copy(col_hbm.at[pl.ds(base, C)], cidx)
            pltpu.async_copy(h_hbm.at[ridx], ra, sem).wait()
            pltpu.async_copy(h_hbm.at[cidx], rb, sem).wait()

            @pl.when(j < ns_chunks)
            def _lds():
                pltpu.sync_copy(s_hbm.at[0, pl.ds(base, C)], sv0)
                pltpu.sync_copy(s_hbm.at[1, pl.ds(base, C)], sv1)

            gate = jnp.full((16,), jnp.where(j < ns_chunks, 1.0, 0.0),
                            jnp.float32)

            def edge(c, ec):
                e0, e1 = ec
                n0 = []
                n1 = []
                for s in range(NSL):
                    pre = ra[c, _sl(s)] + rb[c, _sl(s)] + \
                        (sv0[c, _sl(s)] + sv1[c, _sl(s)]) * gate
                    sg = one / (one + jnp.exp(-pre))
                    ra[c, _sl(s)] = sg
                    n0.append(e0[s] + sg)
                    n1.append(e1[s] + sg * rc[c, _sl(s)])
                return (tuple(n0), tuple(n1))

            pltpu.async_copy(vh_hbm.at[cidx], rc, sem).wait()
            nc = lax.fori_loop(0, C, edge, (ac, at))
            pltpu.sync_copy(ra, sig_hbm.at[pl.ds(base, C)])
            return nc

        a0, a1 = lax.fori_loop(0, t_chunks, chunk,
                               ((zero,) * NSL, (zero,) * NSL))
        for s in range(NSL):
            stage[_sl(s)] = a0[s]
        pltpu.sync_copy(stage, cp_hbm.at[w])
        for s in range(NSL):
            stage[_sl(s)] = a1[s]
        pltpu.sync_copy(stage, tp_hbm.at[w])

    return pl.kernel(
        body,
        out_type=[jax.ShapeDtypeStruct((NW, D), jnp.float32),
                  jax.ShapeDtypeStruct((NW, D), jnp.float32),
                  jax.ShapeDtypeStruct((e_pad, D), jnp.float32)],
        mesh=_MESH(),
        scratch_types=[
            pltpu.VMEM((C,), jnp.int32),
            pltpu.VMEM((C,), jnp.int32),
            pltpu.VMEM((C, D), jnp.float32),
            pltpu.VMEM((C, D), jnp.float32),
            pltpu.VMEM((C, D), jnp.float32),
            pltpu.VMEM((C, D), jnp.float32),
            pltpu.VMEM((C, D), jnp.float32),
            pltpu.VMEM((D,), jnp.float32),
            pltpu.SemaphoreType.DMA,
        ],
    )(row_p, col_p, h_ext, vh_ext, s2core)


def _sc_ct2(col_p, sig, vh_ext, s2core, invc, t_chunks, ns_ext):
    ns_chunks = ns_ext // C

    def body(col_hbm, sig_hbm, vh_hbm, s_hbm, ic_hbm, cp_hbm, tp_hbm,
             cidx, ra, rc, sv0, sv1, icv, stage, sem):
        w = _wid()
        zero = jnp.zeros((16,), jnp.float32)
        one = jnp.full((16,), 1.0, jnp.float32)
        pltpu.sync_copy(ic_hbm, icv)

        def chunk(i, carry):
            ac, at = carry
            j = w + NW * i
            base = j * C
            pltpu.sync_copy(col_hbm.at[pl.ds(base, C)], cidx)
            pltpu.sync_copy(sig_hbm.at[pl.ds(base, C)], ra)
            pltpu.async_copy(vh_hbm.at[cidx], rc, sem).wait()

            @pl.when(j < ns_chunks)
            def _lds():
                pltpu.sync_copy(s_hbm.at[0, pl.ds(base, C)], sv0)
                pltpu.sync_copy(s_hbm.at[1, pl.ds(base, C)], sv1)

            gate = jnp.full((16,), jnp.where(j < ns_chunks, 1.0, 0.0),
                            jnp.float32)

            def edge(c, ec):
                e0, e1 = ec
                n0 = []
                n1 = []
                for s in range(NSL):
                    pre = ra[c, _sl(s)] * icv[_sl(s)] + \
                        (sv0[c, _sl(s)] + sv1[c, _sl(s)]) * gate
                    sg = one / (one + jnp.exp(-pre))
                    n0.append(e0[s] + sg)
                    n1.append(e1[s] + sg * rc[c, _sl(s)])
                return (tuple(n0), tuple(n1))

            return lax.fori_loop(0, C, edge, (ac, at))

        a0, a1 = lax.fori_loop(0, t_chunks, chunk,
                               ((zero,) * NSL, (zero,) * NSL))
        for s in range(NSL):
            stage[_sl(s)] = a0[s]
        pltpu.sync_copy(stage, cp_hbm.at[w])
        for s in range(NSL):
            stage[_sl(s)] = a1[s]
        pltpu.sync_copy(stage, tp_hbm.at[w])

    return pl.kernel(
        body,
        out_type=[jax.ShapeDtypeStruct((NW, D), jnp.float32),
                  jax.ShapeDtypeStruct((NW, D), jnp.float32)],
        mesh=_MESH(),
        scratch_types=[
            pltpu.VMEM((C,), jnp.int32),
            pltpu.VMEM((C, D), jnp.float32),
            pltpu.VMEM((C, D), jnp.float32),
            pltpu.VMEM((C, D), jnp.float32),
            pltpu.VMEM((C, D), jnp.float32),
            pltpu.VMEM((D,), jnp.float32),
            pltpu.VMEM((D,), jnp.float32),
            pltpu.SemaphoreType.DMA,
        ],
    )(col_p, sig, vh_ext, s2core, invc)


_BN = 400


def _tc_mm(x, wt, b, act=None):
    n, k = x.shape
    m = wt.shape[1]

    def body(x_ref, w_ref, b_ref, o_ref):
        y = jnp.dot(x_ref[...], w_ref[...],
                    preferred_element_type=jnp.float32) + b_ref[...]
        if act == "relu":
            y = jnp.maximum(y, 0.0)
        o_ref[...] = y

    return pl.pallas_call(
        body,
        grid=(n // _BN,),
        in_specs=[pl.BlockSpec((_BN, k), lambda i: (i, 0)),
                  pl.BlockSpec((k, m), lambda i: (0, 0)),
                  pl.BlockSpec((1, m), lambda i: (0, 0))],
        out_specs=pl.BlockSpec((_BN, m), lambda i: (i, 0)),
        out_shape=jax.ShapeDtypeStruct((n, m), jnp.float32),
    )(x, wt, b.reshape(1, m))


def _tc_gcnpost(a0, a1, xw):
    n = xw.shape[0]

    def body(a_ref, b_ref, x_ref, o_ref):
        t = jnp.maximum(a_ref[...] + b_ref[...] + x_ref[...], 0.0)
        nrm = jnp.maximum(
            jnp.sqrt(jnp.sum(t * t, axis=1, keepdims=True)), 1e-12)
        o_ref[...] = t / nrm

    return pl.pallas_call(
        body,
        grid=(n // _BN,),
        in_specs=[pl.BlockSpec((_BN, D), lambda i: (i, 0))] * 3,
        out_specs=pl.BlockSpec((_BN, D), lambda i: (i, 0)),
        out_shape=jax.ShapeDtypeStruct((n, D), jnp.float32),
    )(a0, a1, xw)


def _tc_pq(h, en, wpt, bp, wqt, bq, wvt, bv, wut, bu):
    n = h.shape[0]

    def body(h_ref, e_ref, wp_ref, bp_ref, wq_ref, bq_ref, wv_ref, bv_ref,
             wu_ref, bu_ref, p_ref, q_ref, v_ref, u_ref):
        he = jnp.concatenate([h_ref[...], e_ref[...]], axis=1)
        p_ref[...] = jnp.dot(he, wp_ref[...],
                             preferred_element_type=jnp.float32) + bp_ref[...]
        q_ref[...] = jnp.dot(he, wq_ref[...],
                             preferred_element_type=jnp.float32) + bq_ref[...]
        v_ref[...] = jnp.dot(h_ref[...], wv_ref[...],
                             preferred_element_type=jnp.float32) + bv_ref[...]
        u_ref[...] = jnp.dot(h_ref[...], wu_ref[...],
                             preferred_element_type=jnp.float32) + bu_ref[...]

    outs = pl.pallas_call(
        body,
        grid=(n // _BN,),
        in_specs=[pl.BlockSpec((_BN, D), lambda i: (i, 0)),
                  pl.BlockSpec((_BN, D), lambda i: (i, 0)),
                  pl.BlockSpec((2 * D, D), lambda i: (0, 0)),
                  pl.BlockSpec((1, D), lambda i: (0, 0)),
                  pl.BlockSpec((2 * D, D), lambda i: (0, 0)),
                  pl.BlockSpec((1, D), lambda i: (0, 0)),
                  pl.BlockSpec((D, D), lambda i: (0, 0)),
                  pl.BlockSpec((1, D), lambda i: (0, 0)),
                  pl.BlockSpec((D, D), lambda i: (0, 0)),
                  pl.BlockSpec((1, D), lambda i: (0, 0))],
        out_specs=[pl.BlockSpec((_BN, D), lambda i: (i, 0))] * 4,
        out_shape=[jax.ShapeDtypeStruct((n, D), jnp.float32)] * 4,
    )(h, en, wpt, bp.reshape(1, D), wqt, bq.reshape(1, D),
      wvt, bv.reshape(1, D), wut, bu.reshape(1, D))
    return outs


def _tc_pq2(hu1, fin1, sig_head, wpt, bp, wqt, bq, wvt, bv, wut, bu):
    n = hu1.shape[0]

    def body(hu_ref, f_ref, sg_ref, wp_ref, bp_ref, wq_ref, bq_ref,
             wv_ref, bv_ref, wu_ref, bu_ref, p_ref, q_ref, v_ref, u_ref):
        h = jnp.maximum(hu_ref[...] + f_ref[0, :][None, :], 0.0)
        en = sg_ref[...] * f_ref[1, :][None, :]
        he = jnp.concatenate([h, en], axis=1)
        p_ref[...] = jnp.dot(he, wp_ref[...],
                             preferred_element_type=jnp.float32) + bp_ref[...]
        q_ref[...] = jnp.dot(he, wq_ref[...],
                             preferred_element_type=jnp.float32) + bq_ref[...]
        v_ref[...] = jnp.dot(h, wv_ref[...],
                             preferred_element_type=jnp.float32) + bv_ref[...]
        u_ref[...] = jnp.dot(h, wu_ref[...],
                             preferred_element_type=jnp.float32) + bu_ref[...]

    outs = pl.pallas_call(
        body,
        grid=(n // _BN,),
        in_specs=[pl.BlockSpec((_BN, D), lambda i: (i, 0)),
                  pl.BlockSpec((2, D), lambda i: (0, 0)),
                  pl.BlockSpec((_BN, D), lambda i: (i, 0)),
                  pl.BlockSpec((2 * D, D), lambda i: (0, 0)),
                  pl.BlockSpec((1, D), lambda i: (0, 0)),
                  pl.BlockSpec((2 * D, D), lambda i: (0, 0)),
                  pl.BlockSpec((1, D), lambda i: (0, 0)),
                  pl.BlockSpec((D, D), lambda i: (0, 0)),
                  pl.BlockSpec((1, D), lambda i: (0, 0)),
                  pl.BlockSpec((D, D), lambda i: (0, 0)),
                  pl.BlockSpec((1, D), lambda i: (0, 0))],
        out_specs=[pl.BlockSpec((_BN, D), lambda i: (i, 0))] * 4,
        out_shape=[jax.ShapeDtypeStruct((n, D), jnp.float32)] * 4,
    )(hu1, fin1, sig_head, wpt, bp.reshape(1, D), wqt, bq.reshape(1, D),
      wvt, bv.reshape(1, D), wut, bu.reshape(1, D))
    return outs


def _tc_statsfin(sum_p, sq_p, bn_g, bn_b, e_real):
    def body(s_ref, q_ref, g_ref, b_ref, o_ref):
        tot = jnp.sum(s_ref[...], axis=0)
        totsq = jnp.sum(q_ref[...], axis=0)
        mu = tot / e_real
        var = totsq / e_real - mu * mu
        s = g_ref[0, :] * jax.lax.rsqrt(var + 1e-05)
        o_ref[0, :] = s
        o_ref[1, :] = b_ref[0, :] - mu * s

    return pl.pallas_call(
        body,
        out_shape=jax.ShapeDtypeStruct((2, D), jnp.float32),
    )(sum_p, sq_p, bn_g.reshape(1, D), bn_b.reshape(1, D))


def _tc_ctfin(c_p, t_p, n_pad, prev_fin):
    ins = [c_p, t_p]
    if prev_fin is not None:
        ins.append(prev_fin)

    def body(*refs):
        c_ref, t_ref = refs[0], refs[1]
        o_ref = refs[-1]
        if prev_fin is not None:
            f_ref = refs[2]
            corr = n_pad * (1.0 / (1.0 + jnp.exp(-0.5 * f_ref[1, :])))
        else:
            corr = jnp.full((D,), 0.5 * n_pad, jnp.float32)
        c = jnp.sum(c_ref[...], axis=0) - corr + EPS
        o_ref[0, :] = jnp.sum(t_ref[...], axis=0) / c
        o_ref[1, :] = 1.0 / c

    return pl.pallas_call(
        body,
        out_shape=jax.ShapeDtypeStruct((2, D), jnp.float32),
    )(*ins)


def _tc_final(hu2, fin2, cw, cb):
    n = hu2.shape[0]
    nb = n // _BN
    nc = cw.shape[0]

    def body(h_ref, f_ref, w_ref, b_ref, o_ref, acc):
        i = pl.program_id(0)

        @pl.when(i == 0)
        def _init():
            acc[...] = jnp.zeros_like(acc)

        t = jnp.maximum(h_ref[...] + f_ref[0, :][None, :], 0.0)
        acc[...] += jnp.sum(t, axis=0, keepdims=True)

        @pl.when(i == nb - 1)
        def _fin():
            g = acc[0, :] / n
            logits = jnp.sum(g[None, :] * w_ref[...], axis=1) + b_ref[0, :nc]
            m = jnp.max(logits)
            lse = jnp.log(jnp.sum(jnp.exp(logits - m))) + m
            o_ref[...] = jnp.concatenate(
                [logits - lse, jnp.zeros((D - nc,), jnp.float32)]
            ).reshape(1, D)

    out = pl.pallas_call(
        body,
        grid=(nb,),
        in_specs=[pl.BlockSpec((_BN, D), lambda i: (i, 0)),
                  pl.BlockSpec((2, D), lambda i: (0, 0)),
                  pl.BlockSpec((nc, D), lambda i: (0, 0)),
                  pl.BlockSpec((1, D), lambda i: (0, 0))],
        out_specs=pl.BlockSpec((1, D), lambda i: (0, 0)),
        out_shape=jax.ShapeDtypeStruct((1, D), jnp.float32),
        scratch_shapes=[pltpu.VMEM((1, D), jnp.float32)],
    )(hu2, fin2, cw, jnp.pad(cb, (0, D - nc)).reshape(1, D))
    return out[0, :nc]


def _padrow(a):
    return jnp.pad(a, ((0, 1), (0, 0)))


def kernel(x, edge_index, edge_attr, params):
    n, d = x.shape
    e = edge_index.shape[1]
    t_chunks = -(-e // (NW * C))
    e_pad = t_chunks * NW * C
    n_pad = e_pad - e
    ns_ext = (-(-n // C)) * C
    acc_rows = ns_ext + C

    row_p = jnp.concatenate(
        [edge_index[0], jnp.full((n_pad,), n, jnp.int32)])
    col_p = jnp.concatenate(
        [edge_index[1], jnp.full((n_pad,), n, jnp.int32)])
    row_scat = jnp.concatenate(
        [edge_index[0], jnp.full((n_pad,), acc_rows - 1, jnp.int32)])
    ew_p = jnp.concatenate([edge_attr, jnp.zeros((n_pad,), jnp.float32)])
    zeros_big = jnp.zeros((acc_rows, D), jnp.float32)

    p = params
    g1, g2 = p['g1'], p['g2']

    xw1 = _tc_mm(x, p['gcn1']['w'].T, p['gcn1']['b'])
    acc1 = _sc_gcn(row_p, col_p, ew_p, _padrow(xw1), zeros_big,
                   n, t_chunks, acc_rows)
    h1 = _tc_gcnpost(acc1[0], acc1[1], xw1)
    xw2 = _tc_mm(h1, p['gcn2']['w'].T, p['gcn2']['b'])
    acc2 = _sc_gcn(row_p, col_p, ew_p, _padrow(xw2), zeros_big,
                   n, t_chunks, acc_rows)
    h = _tc_gcnpost(acc2[0], acc2[1], xw2)
    h_ext = _padrow(h)

    en1 = _sc_enodes(row_p, col_p, h_ext, ns_ext)[:n]

    wpt1 = jnp.concatenate([g1['A_w'].T, g1['D_w'].T], axis=0)
    wqt1 = jnp.concatenate([g1['B_w'].T, g1['C_w'].T], axis=0)
    p1, q1, vh1, hu1 = _tc_pq(h, en1, wpt1, g1['A_b'] + g1['D_b'],
                              wqt1, g1['B_b'] + g1['C_b'],
                              g1['V_w'].T, g1['V_b'], g1['U_w'].T, g1['U_b'])
    sum1, sq1 = _sc_stats(row_p, col_p, _padrow(p1), _padrow(q1), t_chunks)
    sqb1 = _tc_statsfin(sum1, sq1, g1['bn_g'], g1['bn_b'], float(e))
    s1 = _sc_scatter(row_p, col_p, row_scat, _padrow(p1), _padrow(q1),
                     sqb1, zeros_big, t_chunks, ns_ext, acc_rows)
    c1p, t1p, sig = _sc_ct1(row_p, col_p, h_ext, _padrow(vh1), s1,
                            t_chunks, ns_ext, e_pad)
    fin1 = _tc_ctfin(c1p, t1p, float(n_pad), None)

    wpt2 = jnp.concatenate([g2['A_w'].T, g2['D_w'].T], axis=0)
    wqt2 = jnp.concatenate([g2['B_w'].T, g2['C_w'].T], axis=0)
    p2, q2, vh2, hu2 = _tc_pq2(hu1, fin1, sig[:n], wpt2,
                               g2['A_b'] + g2['D_b'], wqt2,
                               g2['B_b'] + g2['C_b'], g2['V_w'].T, g2['V_b'],
                               g2['U_w'].T, g2['U_b'])
    sum2, sq2 = _sc_stats(row_p, col_p, _padrow(p2), _padrow(q2), t_chunks)
    sqb2 = _tc_statsfin(sum2, sq2, g2['bn_g'], g2['bn_b'], float(e))
    s2 = _sc_scatter(row_p, col_p, row_scat, _padrow(p2), _padrow(q2),
                     sqb2, zeros_big, t_chunks, ns_ext, acc_rows)
    c2p, t2p = _sc_ct2(col_p, sig, _padrow(vh2), s2, fin1[1, :],
                       t_chunks, ns_ext)
    fin2 = _tc_ctfin(c2p, t2p, float(n_pad), fin1)

    return _tc_final(hu2, fin2, p['cls']['w'], p['cls']['b'])


# speedup vs baseline: 3.2046x; 1.2067x over previous
import functools

import jax
import jax.numpy as jnp
from jax import lax
from jax.experimental import pallas as pl
from jax.experimental.pallas import tpu as pltpu
from jax.experimental.pallas import tpu_sc as plsc

NCORE = 2
NSUB = 16
NW = NCORE * NSUB
C = 128
D = 128
NSL = D // 16
EPS = 1e-05

_MESH = functools.partial(
    plsc.VectorSubcoreMesh, core_axis_name="c", subcore_axis_name="s",
    num_cores=NCORE, num_subcores=NSUB)


def _wid():
    return lax.axis_index("s") * NCORE + lax.axis_index("c")


def _sl(i):
    return pl.ds(i * 16, 16)


def _sc_gcn(row_p, col_p, ew_p, xw_ext, zeros_big, n, t_chunks, acc_rows):
    def body(row_hbm, col_hbm, ew_hbm, xw_hbm, z_hbm, out_hbm,
             sidx, didx, ewv, rows, acc_sh, sem):
        cid = lax.axis_index("c")
        sid = lax.axis_index("s")
        w = _wid()

        @pl.when(sid == 0)
        def _zero():
            pltpu.sync_copy(z_hbm, acc_sh)

        plsc.subcore_barrier()

        def chunk(i, _):
            base = (w + NW * i) * C
            h1 = pltpu.async_copy(row_hbm.at[pl.ds(base, C)], sidx, sem)
            h2 = pltpu.async_copy(col_hbm.at[pl.ds(base, C)], didx, sem)
            h3 = pltpu.async_copy(ew_hbm.at[pl.ds(base, C)], ewv, sem)
            h1.wait()
            h2.wait()
            h3.wait()
            pltpu.async_copy(xw_hbm.at[sidx], rows, sem).wait()

            def grp(g, _g):
                wvec = ewv[pl.ds(g * 16, 16)]
                for l in range(16):
                    wv = jnp.full((16,), wvec[l], jnp.float32)
                    c = g * 16 + l
                    for s in range(NSL):
                        rows[c, _sl(s)] = rows[c, _sl(s)] * wv
                return 0

            lax.fori_loop(0, C // 16, grp, 0)
            pltpu.sync_copy(rows, acc_sh.at[didx], add=True)
            return 0

        lax.fori_loop(0, t_chunks, chunk, 0)
        plsc.subcore_barrier()

        @pl.when(sid == 0)
        def _out():
            pltpu.sync_copy(acc_sh.at[pl.ds(0, n)], out_hbm.at[cid])

    return pl.kernel(
        body,
        out_type=jax.ShapeDtypeStruct((NCORE, n, D), jnp.float32),
        mesh=_MESH(),
        scratch_types=[
            pltpu.VMEM((C,), jnp.int32),
            pltpu.VMEM((C,), jnp.int32),
            pltpu.VMEM((C,), jnp.float32),
            pltpu.VMEM((C, D), jnp.float32),
            pltpu.VMEM_SHARED((acc_rows, D), jnp.float32),
            pltpu.SemaphoreType.DMA,
        ],
    )(row_p, col_p, ew_p, xw_ext, zeros_big)


def _sc_enodes(row_p, col_p, h_ext, ns_ext):
    n_chunks = ns_ext // C

    def body(row_hbm, col_hbm, h_hbm, out_hbm, ridx, cidx, ra, rb, sem):
        w = _wid()

        def chunk(i, _):
            j = w + NW * i

            @pl.when(j < n_chunks)
            def _do():
                base = j * C
                h1 = pltpu.async_copy(row_hbm.at[pl.ds(base, C)], ridx, sem)
                h2 = pltpu.async_copy(col_hbm.at[pl.ds(base, C)], cidx, sem)
                h1.wait()
                h2.wait()
                g1 = pltpu.async_copy(h_hbm.at[ridx], ra, sem)
                g2 = pltpu.async_copy(h_hbm.at[cidx], rb, sem)
                g1.wait()
                g2.wait()

                def edge(c, _c):
                    for s in range(NSL):
                        ra[c, _sl(s)] = ra[c, _sl(s)] + rb[c, _sl(s)]
                    return 0

                lax.fori_loop(0, C, edge, 0)
                pltpu.sync_copy(ra, out_hbm.at[pl.ds(base, C)])

            return 0

        lax.fori_loop(0, (n_chunks + NW - 1) // NW, chunk, 0)

    return pl.kernel(
        body,
        out_type=jax.ShapeDtypeStruct((ns_ext, D), jnp.float32),
        mesh=_MESH(),
        scratch_types=[
            pltpu.VMEM((C,), jnp.int32),
            pltpu.VMEM((C,), jnp.int32),
            pltpu.VMEM((C, D), jnp.float32),
            pltpu.VMEM((C, D), jnp.float32),
            pltpu.SemaphoreType.DMA,
        ],
    )(row_p, col_p, h_ext)


def _sc_stats(row_p, col_p, p_ext, q_ext, t_chunks):
    def body(row_hbm, col_hbm, p_hbm, q_hbm, sum_hbm, sq_hbm,
             ridx, cidx, ra, rb, stage, sem):
        w = _wid()
        zero = jnp.zeros((16,), jnp.float32)
        acc0 = (zero,) * NSL
        acc1 = (zero,) * NSL

        def chunk(i, carry):
            a0, a1 = carry
            base = (w + NW * i) * C
            h1 = pltpu.async_copy(row_hbm.at[pl.ds(base, C)], ridx, sem)
            h2 = pltpu.async_copy(col_hbm.at[pl.ds(base, C)], cidx, sem)
            h1.wait()
            h2.wait()
            g1 = pltpu.async_copy(p_hbm.at[ridx], ra, sem)
            g2 = pltpu.async_copy(q_hbm.at[cidx], rb, sem)
            g1.wait()
            g2.wait()

            def edge(c, ec):
                e0, e1 = ec
                n0 = []
                n1 = []
                for s in range(NSL):
                    v = ra[c, _sl(s)] + rb[c, _sl(s)]
                    n0.append(e0[s] + v)
                    n1.append(e1[s] + v * v)
                return (tuple(n0), tuple(n1))

            return lax.fori_loop(0, C, edge, (a0, a1))

        a0, a1 = lax.fori_loop(0, t_chunks, chunk, (acc0, acc1))
        for s in range(NSL):
            stage[_sl(s)] = a0[s]
        pltpu.sync_copy(stage, sum_hbm.at[w])
        for s in range(NSL):
            stage[_sl(s)] = a1[s]
        pltpu.sync_copy(stage, sq_hbm.at[w])

    return pl.kernel(
        body,
        out_type=[jax.ShapeDtypeStruct((NW, D), jnp.float32),
                  jax.ShapeDtypeStruct((NW, D), jnp.float32)],
        mesh=_MESH(),
        scratch_types=[
            pltpu.VMEM((C,), jnp.int32),
            pltpu.VMEM((C,), jnp.int32),
            pltpu.VMEM((C, D), jnp.float32),
            pltpu.VMEM((C, D), jnp.float32),
            pltpu.VMEM((D,), jnp.float32),
            pltpu.SemaphoreType.DMA,
        ],
    )(row_p, col_p, p_ext, q_ext)


def _sc_scatter(row_p, col_p, row_scat, p_ext, q_ext, sqb, zeros_big,
                t_chunks, ns_ext, acc_rows):
    def real_body(row_hbm, col_hbm, rs_hbm, p_hbm, q_hbm, sqb_hbm, z_hbm,
                  out_hbm, ridx, cidx, sidx, ra, rb, coef, acc, sem):
        cid = lax.axis_index("c")
        sid = lax.axis_index("s")
        w = _wid()
        pltpu.sync_copy(sqb_hbm, coef)

        @pl.when(sid == 0)
        def _zero():
            pltpu.sync_copy(z_hbm, acc)

        plsc.subcore_barrier()

        def chunk(i, _):
            base = (w + NW * i) * C
            h1 = pltpu.async_copy(row_hbm.at[pl.ds(base, C)], ridx, sem)
            h2 = pltpu.async_copy(col_hbm.at[pl.ds(base, C)], cidx, sem)
            h3 = pltpu.async_copy(rs_hbm.at[pl.ds(base, C)], sidx, sem)
            h1.wait()
            h2.wait()
            h3.wait()
            g1 = pltpu.async_copy(p_hbm.at[ridx], ra, sem)
            g2 = pltpu.async_copy(q_hbm.at[cidx], rb, sem)
            g1.wait()
            g2.wait()

            def edge(c, _c):
                for s in range(NSL):
                    v = (ra[c, _sl(s)] + rb[c, _sl(s)]) * coef[0, _sl(s)] \
                        + coef[1, _sl(s)]
                    ra[c, _sl(s)] = jnp.maximum(v, 0.0)
                return 0

            lax.fori_loop(0, C, edge, 0)
            pltpu.sync_copy(ra, acc.at[sidx], add=True)
            return 0

        lax.fori_loop(0, t_chunks, chunk, 0)
        plsc.subcore_barrier()

        @pl.when(sid == 0)
        def _out():
            pltpu.sync_copy(acc.at[pl.ds(0, ns_ext)], out_hbm.at[cid])

    return pl.kernel(
        real_body,
        out_type=jax.ShapeDtypeStruct((NCORE, ns_ext, D), jnp.float32),
        mesh=_MESH(),
        scratch_types=[
            pltpu.VMEM((C,), jnp.int32),
            pltpu.VMEM((C,), jnp.int32),
            pltpu.VMEM((C,), jnp.int32),
            pltpu.VMEM((C, D), jnp.float32),
            pltpu.VMEM((C, D), jnp.float32),
            pltpu.VMEM((2, D), jnp.float32),
            pltpu.VMEM_SHARED((acc_rows, D), jnp.float32),
            pltpu.SemaphoreType.DMA,
        ],
    )(row_p, col_p, row_scat, p_ext, q_ext, sqb, zeros_big)


def _sc_ct1(row_p, col_p, h_ext, vh_ext, s2core, t_chunks, ns_ext, e_pad):
    ns_chunks = ns_ext // C

    def body(row_hbm, col_hbm, h_hbm, vh_hbm, s_hbm, cp_hbm, tp_hbm, sig_hbm,
             ridx, cidx, ra, rb, rc, sv0, sv1, stage, sem):
        w = _wid()
        zero = jnp.zeros((16,), jnp.float32)
        one = jnp.full((16,), 1.0, jnp.float32)

        def chunk(i, carry):
            ac, at = carry
            j = w + NW * i
            base = j * C
            h1 = pltpu.async_copy(row_hbm.at[pl.ds(base, C)], ridx, sem)
            h2 = pltpu.async_copy(col_hbm.at[pl.ds(base, C)], cidx, sem)

            @pl.when(j < ns_chunks)
            def _lds():
                s1 = pltpu.async_copy(s_hbm.at[0, pl.ds(base, C)], sv0, sem)
                s2 = pltpu.async_copy(s_hbm.at[1, pl.ds(base, C)], sv1, sem)
                s1.wait()
                s2.wait()

            h1.wait()
            h2.wait()
            g1 = pltpu.async_copy(h_hbm.at[ridx], ra, sem)
            g2 = pltpu.async_copy(h_hbm.at[cidx], rb, sem)
            g3 = pltpu.async_copy(vh_hbm.at[cidx], rc, sem)
            g1.wait()
            g2.wait()
            g3.wait()

            gate = jnp.full((16,), jnp.where(j < ns_chunks, 1.0, 0.0),
                            jnp.float32)

            def edge(c, ec):
                e0, e1 = ec
                n0 = []
                n1 = []
                for s in range(NSL):
                    pre = ra[c, _sl(s)] + rb[c, _sl(s)] + \
                        (sv0[c, _sl(s)] + sv1[c, _sl(s)]) * gate
                    sg = one / (one + jnp.exp(-pre))
                    ra[c, _sl(s)] = sg
                    n0.append(e0[s] + sg)
                    n1.append(e1[s] + sg * rc[c, _sl(s)])
                return (tuple(n0), tuple(n1))

            nc = lax.fori_loop(0, C, edge, (ac, at))
            pltpu.sync_copy(ra, sig_hbm.at[pl.ds(base, C)])
            return nc

        a0, a1 = lax.fori_loop(0, t_chunks, chunk,
                               ((zero,) * NSL, (zero,) * NSL))
        for s in range(NSL):
            stage[_sl(s)] = a0[s]
        pltpu.sync_copy(stage, cp_hbm.at[w])
        for s in range(NSL):
            stage[_sl(s)] = a1[s]
        pltpu.sync_copy(stage, tp_hbm.at[w])

    return pl.kernel(
        body,
        out_type=[jax.ShapeDtypeStruct((NW, D), jnp.float32),
                  jax.ShapeDtypeStruct((NW, D), jnp.float32),
                  jax.ShapeDtypeStruct((e_pad, D), jnp.float32)],
        mesh=_MESH(),
        scratch_types=[
            pltpu.VMEM((C,), jnp.int32),
            pltpu.VMEM((C,), jnp.int32),
            pltpu.VMEM((C, D), jnp.float32),
            pltpu.VMEM((C, D), jnp.float32),
            pltpu.VMEM((C, D), jnp.float32),
            pltpu.VMEM((C, D), jnp.float32),
            pltpu.VMEM((C, D), jnp.float32),
            pltpu.VMEM((D,), jnp.float32),
            pltpu.SemaphoreType.DMA,
        ],
    )(row_p, col_p, h_ext, vh_ext, s2core)


def _sc_ct2(col_p, sig, vh_ext, s2core, invc, t_chunks, ns_ext):
    ns_chunks = ns_ext // C

    def body(col_hbm, sig_hbm, vh_hbm, s_hbm, ic_hbm, cp_hbm, tp_hbm,
             cidx, ra, rc, sv0, sv1, icv, stage, sem):
        w = _wid()
        zero = jnp.zeros((16,), jnp.float32)
        one = jnp.full((16,), 1.0, jnp.float32)
        pltpu.sync_copy(ic_hbm, icv)

        def chunk(i, carry):
            ac, at = carry
            j = w + NW * i
            base = j * C
            h1 = pltpu.async_copy(col_hbm.at[pl.ds(base, C)], cidx, sem)
            h2 = pltpu.async_copy(sig_hbm.at[pl.ds(base, C)], ra, sem)

            @pl.when(j < ns_chunks)
            def _lds():
                s1 = pltpu.async_copy(s_hbm.at[0, pl.ds(base, C)], sv0, sem)
                s2 = pltpu.async_copy(s_hbm.at[1, pl.ds(base, C)], sv1, sem)
                s1.wait()
                s2.wait()

            h1.wait()
            h2.wait()
            pltpu.async_copy(vh_hbm.at[cidx], rc, sem).wait()

            gate = jnp.full((16,), jnp.where(j < ns_chunks, 1.0, 0.0),
                            jnp.float32)

            def edge(c, ec):
                e0, e1 = ec
                n0 = []
                n1 = []
                for s in range(NSL):
                    pre = ra[c, _sl(s)] * icv[_sl(s)] + \
                        (sv0[c, _sl(s)] + sv1[c, _sl(s)]) * gate
                    sg = one / (one + jnp.exp(-pre))
                    n0.append(e0[s] + sg)
                    n1.append(e1[s] + sg * rc[c, _sl(s)])
                return (tuple(n0), tuple(n1))

            return lax.fori_loop(0, C, edge, (ac, at))

        a0, a1 = lax.fori_loop(0, t_chunks, chunk,
                               ((zero,) * NSL, (zero,) * NSL))
        for s in range(NSL):
            stage[_sl(s)] = a0[s]
        pltpu.sync_copy(stage, cp_hbm.at[w])
        for s in range(NSL):
            stage[_sl(s)] = a1[s]
        pltpu.sync_copy(stage, tp_hbm.at[w])

    return pl.kernel(
        body,
        out_type=[jax.ShapeDtypeStruct((NW, D), jnp.float32),
                  jax.ShapeDtypeStruct((NW, D), jnp.float32)],
        mesh=_MESH(),
        scratch_types=[
            pltpu.VMEM((C,), jnp.int32),
            pltpu.VMEM((C, D), jnp.float32),
            pltpu.VMEM((C, D), jnp.float32),
            pltpu.VMEM((C, D), jnp.float32),
            pltpu.VMEM((C, D), jnp.float32),
            pltpu.VMEM((D,), jnp.float32),
            pltpu.VMEM((D,), jnp.float32),
            pltpu.SemaphoreType.DMA,
        ],
    )(col_p, sig, vh_ext, s2core, invc)


_BN = 400


def _tc_mm(x, wt, b, act=None):
    n, k = x.shape
    m = wt.shape[1]

    def body(x_ref, w_ref, b_ref, o_ref):
        y = jnp.dot(x_ref[...], w_ref[...],
                    preferred_element_type=jnp.float32) + b_ref[...]
        if act == "relu":
            y = jnp.maximum(y, 0.0)
        o_ref[...] = y

    return pl.pallas_call(
        body,
        grid=(n // _BN,),
        in_specs=[pl.BlockSpec((_BN, k), lambda i: (i, 0)),
                  pl.BlockSpec((k, m), lambda i: (0, 0)),
                  pl.BlockSpec((1, m), lambda i: (0, 0))],
        out_specs=pl.BlockSpec((_BN, m), lambda i: (i, 0)),
        out_shape=jax.ShapeDtypeStruct((n, m), jnp.float32),
    )(x, wt, b.reshape(1, m))


def _tc_gcnpost(a0, a1, xw):
    n = xw.shape[0]

    def body(a_ref, b_ref, x_ref, o_ref):
        t = jnp.maximum(a_ref[...] + b_ref[...] + x_ref[...], 0.0)
        nrm = jnp.maximum(
            jnp.sqrt(jnp.sum(t * t, axis=1, keepdims=True)), 1e-12)
        o_ref[...] = t / nrm

    return pl.pallas_call(
        body,
        grid=(n // _BN,),
        in_specs=[pl.BlockSpec((_BN, D), lambda i: (i, 0))] * 3,
        out_specs=pl.BlockSpec((_BN, D), lambda i: (i, 0)),
        out_shape=jax.ShapeDtypeStruct((n, D), jnp.float32),
    )(a0, a1, xw)


def _tc_pq(h, en, wpt, bp, wqt, bq, wvt, bv, wut, bu):
    n = h.shape[0]

    def body(h_ref, e_ref, wp_ref, bp_ref, wq_ref, bq_ref, wv_ref, bv_ref,
             wu_ref, bu_ref, p_ref, q_ref, v_ref, u_ref):
        he = jnp.concatenate([h_ref[...], e_ref[...]], axis=1)
        p_ref[...] = jnp.dot(he, wp_ref[...],
                             preferred_element_type=jnp.float32) + bp_ref[...]
        q_ref[...] = jnp.dot(he, wq_ref[...],
                             preferred_element_type=jnp.float32) + bq_ref[...]
        v_ref[...] = jnp.dot(h_ref[...], wv_ref[...],
                             preferred_element_type=jnp.float32) + bv_ref[...]
        u_ref[...] = jnp.dot(h_ref[...], wu_ref[...],
                             preferred_element_type=jnp.float32) + bu_ref[...]

    outs = pl.pallas_call(
        body,
        grid=(n // _BN,),
        in_specs=[pl.BlockSpec((_BN, D), lambda i: (i, 0)),
                  pl.BlockSpec((_BN, D), lambda i: (i, 0)),
                  pl.BlockSpec((2 * D, D), lambda i: (0, 0)),
                  pl.BlockSpec((1, D), lambda i: (0, 0)),
                  pl.BlockSpec((2 * D, D), lambda i: (0, 0)),
                  pl.BlockSpec((1, D), lambda i: (0, 0)),
                  pl.BlockSpec((D, D), lambda i: (0, 0)),
                  pl.BlockSpec((1, D), lambda i: (0, 0)),
                  pl.BlockSpec((D, D), lambda i: (0, 0)),
                  pl.BlockSpec((1, D), lambda i: (0, 0))],
        out_specs=[pl.BlockSpec((_BN, D), lambda i: (i, 0))] * 4,
        out_shape=[jax.ShapeDtypeStruct((n, D), jnp.float32)] * 4,
    )(h, en, wpt, bp.reshape(1, D), wqt, bq.reshape(1, D),
      wvt, bv.reshape(1, D), wut, bu.reshape(1, D))
    return outs


def _tc_pq2(hu1, fin1, sig_head, wpt, bp, wqt, bq, wvt, bv, wut, bu):
    n = hu1.shape[0]

    def body(hu_ref, f_ref, sg_ref, wp_ref, bp_ref, wq_ref, bq_ref,
             wv_ref, bv_ref, wu_ref, bu_ref, p_ref, q_ref, v_ref, u_ref):
        h = jnp.maximum(hu_ref[...] + f_ref[0, :][None, :], 0.0)
        en = sg_ref[...] * f_ref[1, :][None, :]
        he = jnp.concatenate([h, en], axis=1)
        p_ref[...] = jnp.dot(he, wp_ref[...],
                             preferred_element_type=jnp.float32) + bp_ref[...]
        q_ref[...] = jnp.dot(he, wq_ref[...],
                             preferred_element_type=jnp.float32) + bq_ref[...]
        v_ref[...] = jnp.dot(h, wv_ref[...],
                             preferred_element_type=jnp.float32) + bv_ref[...]
        u_ref[...] = jnp.dot(h, wu_ref[...],
                             preferred_element_type=jnp.float32) + bu_ref[...]

    outs = pl.pallas_call(
        body,
        grid=(n // _BN,),
        in_specs=[pl.BlockSpec((_BN, D), lambda i: (i, 0)),
                  pl.BlockSpec((2, D), lambda i: (0, 0)),
                  pl.BlockSpec((_BN, D), lambda i: (i, 0)),
                  pl.BlockSpec((2 * D, D), lambda i: (0, 0)),
                  pl.BlockSpec((1, D), lambda i: (0, 0)),
                  pl.BlockSpec((2 * D, D), lambda i: (0, 0)),
                  pl.BlockSpec((1, D), lambda i: (0, 0)),
                  pl.BlockSpec((D, D), lambda i: (0, 0)),
                  pl.BlockSpec((1, D), lambda i: (0, 0)),
                  pl.BlockSpec((D, D), lambda i: (0, 0)),
                  pl.BlockSpec((1, D), lambda i: (0, 0))],
        out_specs=[pl.BlockSpec((_BN, D), lambda i: (i, 0))] * 4,
        out_shape=[jax.ShapeDtypeStruct((n, D), jnp.float32)] * 4,
    )(hu1, fin1, sig_head, wpt, bp.reshape(1, D), wqt, bq.reshape(1, D),
      wvt, bv.reshape(1, D), wut, bu.reshape(1, D))
    return outs


def _tc_statsfin(sum_p, sq_p, bn_g, bn_b, e_real):
    def body(s_ref, q_ref, g_ref, b_ref, o_ref):
        tot = jnp.sum(s_ref[...], axis=0)
        totsq = jnp.sum(q_ref[...], axis=0)
        mu = tot / e_real
        var = totsq / e_real - mu * mu
        s = g_ref[0, :] * jax.lax.rsqrt(var + 1e-05)
        o_ref[0, :] = s
        o_ref[1, :] = b_ref[0, :] - mu * s

    return pl.pallas_call(
        body,
        out_shape=jax.ShapeDtypeStruct((2, D), jnp.float32),
    )(sum_p, sq_p, bn_g.reshape(1, D), bn_b.reshape(1, D))


def _tc_ctfin(c_p, t_p, n_pad, prev_fin):
    ins = [c_p, t_p]
    if prev_fin is not None:
        ins.append(prev_fin)

    def body(*refs):
        c_ref, t_ref = refs[0], refs[1]
        o_ref = refs[-1]
        if prev_fin is not None:
            f_ref = refs[2]
            corr = n_pad * (1.0 / (1.0 + jnp.exp(-0.5 * f_ref[1, :])))
        else:
            corr = jnp.full((D,), 0.5 * n_pad, jnp.float32)
        c = jnp.sum(c_ref[...], axis=0) - corr + EPS
        o_ref[0, :] = jnp.sum(t_ref[...], axis=0) / c
        o_ref[1, :] = 1.0 / c

    return pl.pallas_call(
        body,
        out_shape=jax.ShapeDtypeStruct((2, D), jnp.float32),
    )(*ins)


def _tc_final(hu2, fin2, cw, cb):
    n = hu2.shape[0]
    nb = n // _BN
    nc = cw.shape[0]

    def body(h_ref, f_ref, w_ref, b_ref, o_ref, acc):
        i = pl.program_id(0)

        @pl.when(i == 0)
        def _init():
            acc[...] = jnp.zeros_like(acc)

        t = jnp.maximum(h_ref[...] + f_ref[0, :][None, :], 0.0)
        acc[...] += jnp.sum(t, axis=0, keepdims=True)

        @pl.when(i == nb - 1)
        def _fin():
            g = acc[0, :] / n
            logits = jnp.sum(g[None, :] * w_ref[...], axis=1) + b_ref[0, :nc]
            m = jnp.max(logits)
            lse = jnp.log(jnp.sum(jnp.exp(logits - m))) + m
            o_ref[...] = jnp.concatenate(
                [logits - lse, jnp.zeros((D - nc,), jnp.float32)]
            ).reshape(1, D)

    out = pl.pallas_call(
        body,
        grid=(nb,),
        in_specs=[pl.BlockSpec((_BN, D), lambda i: (i, 0)),
                  pl.BlockSpec((2, D), lambda i: (0, 0)),
                  pl.BlockSpec((nc, D), lambda i: (0, 0)),
                  pl.BlockSpec((1, D), lambda i: (0, 0))],
        out_specs=pl.BlockSpec((1, D), lambda i: (0, 0)),
        out_shape=jax.ShapeDtypeStruct((1, D), jnp.float32),
        scratch_shapes=[pltpu.VMEM((1, D), jnp.float32)],
    )(hu2, fin2, cw, jnp.pad(cb, (0, D - nc)).reshape(1, D))
    return out[0, :nc]


def _padrow(a):
    return jnp.pad(a, ((0, 1), (0, 0)))


def kernel(x, edge_index, edge_attr, params):
    n, d = x.shape
    e = edge_index.shape[1]
    t_chunks = -(-e // (NW * C))
    e_pad = t_chunks * NW * C
    n_pad = e_pad - e
    ns_ext = (-(-n // C)) * C
    acc_rows = ns_ext + C

    row_p = jnp.concatenate(
        [edge_index[0], jnp.full((n_pad,), n, jnp.int32)])
    col_p = jnp.concatenate(
        [edge_index[1], jnp.full((n_pad,), n, jnp.int32)])
    row_scat = jnp.concatenate(
        [edge_index[0], jnp.full((n_pad,), acc_rows - 1, jnp.int32)])
    ew_p = jnp.concatenate([edge_attr, jnp.zeros((n_pad,), jnp.float32)])
    zeros_big = jnp.zeros((acc_rows, D), jnp.float32)

    p = params
    g1, g2 = p['g1'], p['g2']

    xw1 = _tc_mm(x, p['gcn1']['w'].T, p['gcn1']['b'])
    acc1 = _sc_gcn(row_p, col_p, ew_p, _padrow(xw1), zeros_big,
                   n, t_chunks, acc_rows)
    h1 = _tc_gcnpost(acc1[0], acc1[1], xw1)
    xw2 = _tc_mm(h1, p['gcn2']['w'].T, p['gcn2']['b'])
    acc2 = _sc_gcn(row_p, col_p, ew_p, _padrow(xw2), zeros_big,
                   n, t_chunks, acc_rows)
    h = _tc_gcnpost(acc2[0], acc2[1], xw2)
    h_ext = _padrow(h)

    en1 = _sc_enodes(row_p, col_p, h_ext, ns_ext)[:n]

    wpt1 = jnp.concatenate([g1['A_w'].T, g1['D_w'].T], axis=0)
    wqt1 = jnp.concatenate([g1['B_w'].T, g1['C_w'].T], axis=0)
    p1, q1, vh1, hu1 = _tc_pq(h, en1, wpt1, g1['A_b'] + g1['D_b'],
                              wqt1, g1['B_b'] + g1['C_b'],
                              g1['V_w'].T, g1['V_b'], g1['U_w'].T, g1['U_b'])
    sum1, sq1 = _sc_stats(row_p, col_p, _padrow(p1), _padrow(q1), t_chunks)
    sqb1 = _tc_statsfin(sum1, sq1, g1['bn_g'], g1['bn_b'], float(e))
    s1 = _sc_scatter(row_p, col_p, row_scat, _padrow(p1), _padrow(q1),
                     sqb1, zeros_big, t_chunks, ns_ext, acc_rows)
    c1p, t1p, sig = _sc_ct1(row_p, col_p, h_ext, _padrow(vh1), s1,
                            t_chunks, ns_ext, e_pad)
    fin1 = _tc_ctfin(c1p, t1p, float(n_pad), None)

    wpt2 = jnp.concatenate([g2['A_w'].T, g2['D_w'].T], axis=0)
    wqt2 = jnp.concatenate([g2['B_w'].T, g2['C_w'].T], axis=0)
    p2, q2, vh2, hu2 = _tc_pq2(hu1, fin1, sig[:n], wpt2,
                               g2['A_b'] + g2['D_b'], wqt2,
                               g2['B_b'] + g2['C_b'], g2['V_w'].T, g2['V_b'],
                               g2['U_w'].T, g2['U_b'])
    sum2, sq2 = _sc_stats(row_p, col_p, _padrow(p2), _padrow(q2), t_chunks)
    sqb2 = _tc_statsfin(sum2, sq2, g2['bn_g'], g2['bn_b'], float(e))
    s2 = _sc_scatter(row_p, col_p, row_scat, _padrow(p2), _padrow(q2),
                     sqb2, zeros_big, t_chunks, ns_ext, acc_rows)
    c2p, t2p = _sc_ct2(col_p, sig, _padrow(vh2), s2, fin1[1, :],
                       t_chunks, ns_ext)
    fin2 = _tc_ctfin(c2p, t2p, float(n_pad), fin1)

    return _tc_final(hu2, fin2, p['cls']['w'], p['cls']['b'])
